# Initial kernel scaffold; baseline (speedup 1.0000x reference)
import jax
import jax.numpy as jnp
from jax.experimental import pallas as pl


def kernel(input_tensor, weight):
    raise NotImplementedError("write your pallas kernel here")


# speedup vs baseline: 3.4193x; 3.4193x over previous
import jax
import jax.numpy as jnp
from jax.experimental import pallas as pl


def _body(w_ref, o_ref):
    o_ref[...] = w_ref[...][None]


def kernel(input_tensor, weight):
    bsz, seq_len, dim = input_tensor.shape
    blk = 512
    table = weight[:seq_len]
    return pl.pallas_call(
        _body,
        grid=(seq_len // blk, bsz),
        in_specs=[pl.BlockSpec((blk, dim), lambda i, j: (i, 0))],
        out_specs=pl.BlockSpec((1, blk, dim), lambda i, j: (j, i, 0)),
        out_shape=jax.ShapeDtypeStruct((bsz, seq_len, dim), weight.dtype),
    )(table)


# speedup vs baseline: 4.2248x; 1.2356x over previous
import jax
import jax.numpy as jnp
from jax.experimental import pallas as pl


def _body(w_ref, o_ref):
    o_ref[...] = w_ref[...][None]


def kernel(input_tensor, weight):
    bsz, seq_len, dim = input_tensor.shape
    blk = 1024
    table = weight[:seq_len]
    return pl.pallas_call(
        _body,
        grid=(seq_len // blk, bsz),
        in_specs=[pl.BlockSpec((blk, dim), lambda i, j: (i, 0))],
        out_specs=pl.BlockSpec((1, blk, dim), lambda i, j: (j, i, 0)),
        out_shape=jax.ShapeDtypeStruct((bsz, seq_len, dim), weight.dtype),
    )(table)


# speedup vs baseline: 4.6504x; 1.1007x over previous
import jax
import jax.numpy as jnp
from jax.experimental import pallas as pl


def _body(w_ref, o_ref):
    o_ref[...] = w_ref[...][None]


def kernel(input_tensor, weight):
    bsz, seq_len, dim = input_tensor.shape
    blk = 2048
    table = weight[:seq_len]
    return pl.pallas_call(
        _body,
        grid=(seq_len // blk, bsz),
        in_specs=[pl.BlockSpec((blk, dim), lambda i, j: (i, 0))],
        out_specs=pl.BlockSpec((1, blk, dim), lambda i, j: (j, i, 0)),
        out_shape=jax.ShapeDtypeStruct((bsz, seq_len, dim), weight.dtype),
    )(table)
